# baseline (device time: 55626 ns/iter reference)
import jax
import jax.numpy as jnp
from jax import lax
from jax.experimental import pallas as pl
from jax.experimental.pallas import tpu as pltpu

N_DEV = 8
B, SQ, DM, HQ_TOT, DH = 2, 256, 512, 32, 64
H_PER = HQ_TOT // N_DEV
BLK = 64
ROWS = B * SQ
CHUNK = ROWS // N_DEV
N_STEPS = N_DEV - 1


def _body(x_ref, wq_ref, k_ref, v_ref, wo_ref, out_ref,
          ctx_ref, comm_ref, send_sems, recv_sems):
    my = lax.axis_index("i")
    right = lax.rem(my + 1, N_DEV)
    left = lax.rem(my + N_DEV - 1, N_DEV)

    q = jnp.dot(x_ref[...], wq_ref[...], preferred_element_type=jnp.float32)
    q = (q * 0.125).astype(jnp.bfloat16)

    r_blk = lax.broadcasted_iota(jnp.int32, (SQ, SQ), 0) // BLK
    c_blk = lax.broadcasted_iota(jnp.int32, (SQ, SQ), 1) // BLK
    mask = r_blk == c_blk

    for b in range(B):
        for h in range(H_PER):
            qh = q[b * SQ:(b + 1) * SQ, h * DH:(h + 1) * DH]
            scores = lax.dot_general(
                qh, k_ref[h, b], (((1,), (1,)), ((), ())),
                preferred_element_type=jnp.float32)
            scores = jnp.where(mask, scores, -1e9)
            m = jnp.max(scores, axis=1, keepdims=True)
            w = jnp.exp(scores - m)
            w = (w / jnp.sum(w, axis=1, keepdims=True)).astype(jnp.bfloat16)
            ctx = jnp.dot(w, v_ref[h, b], preferred_element_type=jnp.float32)
            ctx_ref[b * SQ:(b + 1) * SQ, h * DH:(h + 1) * DH] = (
                ctx.astype(jnp.bfloat16))

    out_ref[...] = jnp.dot(ctx_ref[...], wo_ref[...],
                           preferred_element_type=jnp.float32)

    barrier = pltpu.get_barrier_semaphore()
    for nbr in (left, right):
        pl.semaphore_signal(barrier, inc=1, device_id=(nbr,),
                            device_id_type=pltpu.DeviceIdType.MESH)
    pl.semaphore_wait(barrier, 2)

    for s in range(N_STEPS):
        send_c = lax.rem(my + (N_DEV - s), N_DEV)
        recv_c = lax.rem(my + (N_DEV - s - 1), N_DEV)
        rdma = pltpu.make_async_remote_copy(
            src_ref=out_ref.at[pl.ds(send_c * CHUNK, CHUNK), :],
            dst_ref=comm_ref.at[s],
            send_sem=send_sems.at[s],
            recv_sem=recv_sems.at[s],
            device_id=(right,),
            device_id_type=pltpu.DeviceIdType.MESH,
        )
        rdma.start()
        rdma.wait()
        rows = pl.ds(recv_c * CHUNK, CHUNK)
        out_ref[rows, :] = out_ref[rows, :] + comm_ref[s]

    for s in range(N_STEPS):
        send_c = lax.rem(my + 1 + (N_DEV - s), N_DEV)
        recv_c = lax.rem(my + (N_DEV - s), N_DEV)
        t = N_STEPS + s
        rdma = pltpu.make_async_remote_copy(
            src_ref=out_ref.at[pl.ds(send_c * CHUNK, CHUNK), :],
            dst_ref=comm_ref.at[t],
            send_sem=send_sems.at[t],
            recv_sem=recv_sems.at[t],
            device_id=(right,),
            device_id_type=pltpu.DeviceIdType.MESH,
        )
        rdma.start()
        rdma.wait()
        out_ref[pl.ds(recv_c * CHUNK, CHUNK), :] = comm_ref[t]


def kernel(x, Wq, K_ext, V_ext, Wo):
    i = lax.axis_index("i")
    xb = x.reshape(ROWS, DM).astype(jnp.bfloat16)
    wq = Wq.astype(jnp.bfloat16)
    wo = Wo.astype(jnp.bfloat16)
    k_t = jnp.transpose(K_ext, (2, 0, 1, 3))
    v_t = jnp.transpose(V_ext, (2, 0, 1, 3))
    k_my = lax.dynamic_slice_in_dim(k_t, i * H_PER, H_PER, 0).astype(jnp.bfloat16)
    v_my = lax.dynamic_slice_in_dim(v_t, i * H_PER, H_PER, 0).astype(jnp.bfloat16)

    out = pl.pallas_call(
        _body,
        out_shape=jax.ShapeDtypeStruct((ROWS, DM), jnp.float32),
        in_specs=[pl.BlockSpec(memory_space=pltpu.VMEM)] * 5,
        out_specs=pl.BlockSpec(memory_space=pltpu.VMEM),
        scratch_shapes=[
            pltpu.VMEM((ROWS, H_PER * DH), jnp.bfloat16),
            pltpu.VMEM((2 * N_STEPS, CHUNK, DM), jnp.float32),
            pltpu.SemaphoreType.DMA((2 * N_STEPS,)),
            pltpu.SemaphoreType.DMA((2 * N_STEPS,)),
        ],
        compiler_params=pltpu.CompilerParams(collective_id=0),
    )(xb, wq, k_my, v_my, wo)
    return out.reshape(B, SQ, DM)


# device time: 22702 ns/iter; 2.4503x vs baseline; 2.4503x over previous
import jax
import jax.numpy as jnp
from jax import lax
from jax.experimental import pallas as pl
from jax.experimental.pallas import tpu as pltpu

N_DEV = 8
B, SQ, DM, HQ_TOT, DH = 2, 256, 512, 32, 64
H_PER = HQ_TOT // N_DEV
BLK = 64
ROWS = B * SQ
CHUNK = ROWS // N_DEV


def _body(x_ref, wq_ref, k_ref, v_ref, wo_ref, out_ref,
          ctx_ref, part_ref, p1_ref, red_ref,
          p1_send, p1_recv, p2_send, p2_recv):
    my = lax.axis_index("i")

    q = jnp.dot(x_ref[...], wq_ref[...], preferred_element_type=jnp.float32)
    q = (q * 0.125).astype(jnp.bfloat16)

    r_blk = lax.broadcasted_iota(jnp.int32, (SQ, SQ), 0) // BLK
    c_blk = lax.broadcasted_iota(jnp.int32, (SQ, SQ), 1) // BLK
    mask = r_blk == c_blk

    for b in range(B):
        for h in range(H_PER):
            qh = q[b * SQ:(b + 1) * SQ, h * DH:(h + 1) * DH]
            scores = lax.dot_general(
                qh, k_ref[h, b], (((1,), (1,)), ((), ())),
                preferred_element_type=jnp.float32)
            scores = jnp.where(mask, scores, -1e9)
            m = jnp.max(scores, axis=1, keepdims=True)
            w = jnp.exp(scores - m)
            w = (w / jnp.sum(w, axis=1, keepdims=True)).astype(jnp.bfloat16)
            ctx = jnp.dot(w, v_ref[h, b], preferred_element_type=jnp.float32)
            ctx_ref[b * SQ:(b + 1) * SQ, h * DH:(h + 1) * DH] = (
                ctx.astype(jnp.bfloat16))

    part_ref[...] = jnp.dot(ctx_ref[...], wo_ref[...],
                            preferred_element_type=jnp.float32
                            ).astype(jnp.bfloat16)

    barrier = pltpu.get_barrier_semaphore()
    for k in range(1, N_DEV):
        pl.semaphore_signal(barrier, inc=1,
                            device_id=(lax.rem(my + k, N_DEV),),
                            device_id_type=pltpu.DeviceIdType.MESH)
    pl.semaphore_wait(barrier, N_DEV - 1)

    p1 = []
    for k in range(1, N_DEV):
        d = lax.rem(my + k, N_DEV)
        rdma = pltpu.make_async_remote_copy(
            src_ref=part_ref.at[pl.ds(d * CHUNK, CHUNK), :],
            dst_ref=p1_ref.at[k - 1],
            send_sem=p1_send.at[k - 1],
            recv_sem=p1_recv.at[k - 1],
            device_id=(d,),
            device_id_type=pltpu.DeviceIdType.MESH,
        )
        rdma.start()
        p1.append(rdma)

    acc = part_ref[pl.ds(my * CHUNK, CHUNK), :].astype(jnp.float32)
    for j in range(N_DEV - 1):
        p1[j].wait_recv()
        acc = acc + p1_ref[j].astype(jnp.float32)
    red_ref[...] = acc.astype(jnp.bfloat16)
    out_ref[pl.ds(my * CHUNK, CHUNK), :] = red_ref[...]

    p2 = []
    for k in range(1, N_DEV):
        d = lax.rem(my + k, N_DEV)
        rdma = pltpu.make_async_remote_copy(
            src_ref=red_ref,
            dst_ref=out_ref.at[pl.ds(my * CHUNK, CHUNK), :],
            send_sem=p2_send.at[k - 1],
            recv_sem=p2_recv.at[k - 1],
            device_id=(d,),
            device_id_type=pltpu.DeviceIdType.MESH,
        )
        rdma.start()
        p2.append(rdma)

    for j in range(N_DEV - 1):
        p1[j].wait_send()

    for j in range(N_DEV - 1):
        sdev = lax.rem(my + N_DEV - (j + 1), N_DEV)
        recv = pltpu.make_async_remote_copy(
            src_ref=red_ref,
            dst_ref=out_ref.at[pl.ds(sdev * CHUNK, CHUNK), :],
            send_sem=p2_send.at[j],
            recv_sem=p2_recv.at[j],
            device_id=(sdev,),
            device_id_type=pltpu.DeviceIdType.MESH,
        )
        recv.wait_recv()

    for j in range(N_DEV - 1):
        p2[j].wait_send()


def kernel(x, Wq, K_ext, V_ext, Wo):
    i = lax.axis_index("i")
    xb = x.reshape(ROWS, DM).astype(jnp.bfloat16)
    wq = Wq.astype(jnp.bfloat16)
    wo = Wo.astype(jnp.bfloat16)
    k_t = jnp.transpose(K_ext, (2, 0, 1, 3))
    v_t = jnp.transpose(V_ext, (2, 0, 1, 3))
    k_my = lax.dynamic_slice_in_dim(k_t, i * H_PER, H_PER, 0).astype(jnp.bfloat16)
    v_my = lax.dynamic_slice_in_dim(v_t, i * H_PER, H_PER, 0).astype(jnp.bfloat16)

    out = pl.pallas_call(
        _body,
        out_shape=jax.ShapeDtypeStruct((ROWS, DM), jnp.bfloat16),
        in_specs=[pl.BlockSpec(memory_space=pltpu.VMEM)] * 5,
        out_specs=pl.BlockSpec(memory_space=pltpu.VMEM),
        scratch_shapes=[
            pltpu.VMEM((ROWS, H_PER * DH), jnp.bfloat16),
            pltpu.VMEM((ROWS, DM), jnp.bfloat16),
            pltpu.VMEM((N_DEV - 1, CHUNK, DM), jnp.bfloat16),
            pltpu.VMEM((CHUNK, DM), jnp.bfloat16),
            pltpu.SemaphoreType.DMA((N_DEV - 1,)),
            pltpu.SemaphoreType.DMA((N_DEV - 1,)),
            pltpu.SemaphoreType.DMA((N_DEV - 1,)),
            pltpu.SemaphoreType.DMA((N_DEV - 1,)),
        ],
        compiler_params=pltpu.CompilerParams(collective_id=0),
    )(xb, wq, k_my, v_my, wo)
    return out.astype(jnp.float32).reshape(B, SQ, DM)
